# baseline (speedup 1.0000x reference)
import functools

import jax
import jax.numpy as jnp
from jax import lax
from jax.experimental import pallas as pl
from jax.experimental.pallas import tpu as pltpu
from jax.experimental.pallas import tpu_sc as plsc

_TIME = 288
_B, _T, _N, _C = 64, 12, 8192, 3
_F = 64
_L = 16
_NC, _NS = 2, 16
_NW = _NC * _NS
_NPW = _N // _NW
_NVEC = _NPW // _L
_WPAD = 8
_CT = _TIME * _WPAD
_QF = 16
_NQ = _F // _QF
_BH = _B // 2


def _tec_body(x_hbm, dayt_hbm, weekt_hbm, out_hbm,
              stage_v, cidx_v, dayt_v, weekt_v, ctab_v, outbuf_v,
              sem0, sem1):
    cid = lax.axis_index("c")
    sid = lax.axis_index("s")
    wid = sid * _NC + cid
    n0 = wid * _NPW

    pltpu.sync_copy(dayt_hbm, dayt_v)
    pltpu.sync_copy(weekt_hbm, weekt_v)

    iota = lax.broadcasted_iota(jnp.int32, (_L,), 0)

    for h in range(2):
        b0 = h * _BH
        pltpu.sync_copy(
            x_hbm.at[pl.ds(b0, _BH), _T - 1, pl.ds(n0 * _C, _NPW * _C)],
            stage_v)

        @plsc.parallel_loop(0, _BH * _NVEC)
        def _idx_body(i, b0=b0):
            bi = i // _NVEC
            j = i - bi * _NVEC
            bvec = jnp.full((_L,), 0, jnp.int32) + bi
            nvec = (j * _L + iota) * _C
            dv = plsc.load_gather(stage_v, [bvec, nvec + 1])
            wv = plsc.load_gather(stage_v, [bvec, nvec + 2])
            d = jnp.clip(
                lax.convert_element_type(dv * float(_TIME), jnp.int32),
                0, _TIME - 1)
            w = jnp.clip(lax.convert_element_type(wv, jnp.int32), 0, 6)
            cidx_v[b0 + bi, pl.ds(j * _L, _L)] = d * _WPAD + w

    wsel = jnp.bitwise_and(iota, _WPAD - 1)
    dsel = lax.shift_right_logical(iota, 3)

    sems = (sem0, sem1)

    for q in range(_NQ):
        f0 = q * _QF

        for fi in range(_QF):
            f = f0 + fi
            wrow = plsc.load_gather(weekt_v, [f * _WPAD + wsel])

            @plsc.parallel_loop(0, _CT // _L)
            def _build_body(j, f=f, fi=fi, wrow=wrow):
                dvals = plsc.load_gather(dayt_v, [f * _TIME + j * 2 + dsel])
                ctab_v[pl.ds(fi * _CT + j * _L, _L)] = dvals + wrow

        def batch_pair(bb, _, f0=f0):
            for k in range(2):
                b = bb * 2 + k

                @pl.when(bb > 0)
                def _wait(k=k, b=b):
                    pltpu.make_async_copy(
                        outbuf_v.at[k],
                        out_hbm.at[b, pl.ds(f0, _QF), pl.ds(n0, _NPW)],
                        sems[k],
                    ).wait()

                @plsc.parallel_loop(0, _NVEC)
                def _vec_body(j, k=k, b=b):
                    cvec = cidx_v[b, pl.ds(j * _L, _L)]
                    for fi in range(_QF):
                        g = plsc.load_gather(ctab_v, [cvec + fi * _CT])
                        outbuf_v[k, fi, pl.ds(j * _L, _L)] = g

                pltpu.async_copy(
                    outbuf_v.at[k],
                    out_hbm.at[b, pl.ds(f0, _QF), pl.ds(n0, _NPW)],
                    sems[k],
                )
            return 0

        lax.fori_loop(0, _B // 2, batch_pair, 0)

        for k in range(2):
            pltpu.make_async_copy(
                outbuf_v.at[k],
                out_hbm.at[_B - 2 + k, pl.ds(f0, _QF), pl.ds(n0, _NPW)],
                sems[k],
            ).wait()


@functools.partial(
    pl.kernel,
    mesh=plsc.VectorSubcoreMesh(core_axis_name="c", subcore_axis_name="s"),
    out_type=jax.ShapeDtypeStruct((_B, _F, _N), jnp.float32),
    compiler_params=pltpu.CompilerParams(needs_layout_passes=False),
    scratch_types=[
        pltpu.VMEM((_BH, _NPW * _C), jnp.float32),
        pltpu.VMEM((_B, _NPW), jnp.int32),
        pltpu.VMEM((_F * _TIME,), jnp.float32),
        pltpu.VMEM((_F * _WPAD,), jnp.float32),
        pltpu.VMEM((_QF * _CT,), jnp.float32),
        pltpu.VMEM((2, _QF, _NPW), jnp.float32),
        pltpu.SemaphoreType.DMA,
        pltpu.SemaphoreType.DMA,
    ],
)
def _sc_lookup(x_hbm, dayt_hbm, weekt_hbm, out_hbm,
               stage_v, cidx_v, dayt_v, weekt_v, ctab_v, outbuf_v,
               sem0, sem1):
    _tec_body(x_hbm, dayt_hbm, weekt_hbm, out_hbm,
              stage_v, cidx_v, dayt_v, weekt_v, ctab_v, outbuf_v,
              sem0, sem1)


def kernel(x, time_day, time_week):
    x2 = x.reshape(_B, _T, _N * _C)
    dayt = jnp.transpose(time_day).reshape(-1)
    weekt = jnp.concatenate(
        [jnp.transpose(time_week),
         jnp.zeros((_F, _WPAD - 7), jnp.float32)], axis=1).reshape(-1)
    out = _sc_lookup(x2, dayt, weekt)
    return out[..., None]


# speedup vs baseline: 2.7298x; 2.7298x over previous
import functools

import jax
import jax.numpy as jnp
from jax import lax
from jax.experimental import pallas as pl
from jax.experimental.pallas import tpu as pltpu
from jax.experimental.pallas import tpu_sc as plsc

_TIME = 288
_B, _T, _N, _C = 64, 12, 8192, 3
_F = 64
_L = 16
_NC, _NS = 2, 16
_NW = _NC * _NS
_NPW = _N // _NW
_NVEC = _NPW // _L
_NB = _N // 128
_NBW = _NPW // 128
_WPAD = 8
_CT = _TIME * _WPAD
_QF = 16
_NQ = _F // _QF


def _tec_body(day_hbm, week_hbm, dayt_hbm, weekt_hbm, out_hbm,
              stage_v, cidx_v, dayt_v, weekt_v, ctab_v, outbuf_v,
              sem0, sem1):
    cid = lax.axis_index("c")
    sid = lax.axis_index("s")
    wid = sid * _NC + cid
    n0 = wid * _NPW
    nb0 = wid * _NBW

    pltpu.sync_copy(dayt_hbm, dayt_v)
    pltpu.sync_copy(weekt_hbm, weekt_v)

    iota = lax.broadcasted_iota(jnp.int32, (_L,), 0)

    pltpu.sync_copy(day_hbm.at[:, pl.ds(n0, _NPW)], stage_v)

    @plsc.parallel_loop(0, _B * _NVEC)
    def _day_idx_body(i):
        b = i // _NVEC
        j = i - b * _NVEC
        v = stage_v[b, pl.ds(j * _L, _L)]
        d = lax.convert_element_type(v * float(_TIME), jnp.int32)
        cidx_v[b, pl.ds(j * _L, _L)] = jnp.clip(d, 0, _TIME - 1) * _WPAD

    pltpu.sync_copy(week_hbm.at[:, pl.ds(n0, _NPW)], stage_v)

    @plsc.parallel_loop(0, _B * _NVEC)
    def _week_idx_body(i):
        b = i // _NVEC
        j = i - b * _NVEC
        v = stage_v[b, pl.ds(j * _L, _L)]
        w = lax.convert_element_type(v, jnp.int32)
        sl = (b, pl.ds(j * _L, _L))
        cidx_v[sl] = cidx_v[sl] + jnp.clip(w, 0, 6)

    wsel = jnp.bitwise_and(iota, _WPAD - 1)
    dsel = lax.shift_right_logical(iota, 3)

    sems = (sem0, sem1)

    for q in range(_NQ):
        f0 = q * _QF

        for fi in range(_QF):
            f = f0 + fi
            wrow = plsc.load_gather(weekt_v, [f * _WPAD + wsel])

            @plsc.parallel_loop(0, _CT // _L)
            def _build_body(j, f=f, fi=fi, wrow=wrow):
                dvals = plsc.load_gather(dayt_v, [f * _TIME + j * 2 + dsel])
                ctab_v[pl.ds(fi * _CT + j * _L, _L)] = dvals + wrow

        def batch_pair(bb, _, f0=f0):
            for k in range(2):
                b = bb * 2 + k

                @pl.when(bb > 0)
                def _wait(k=k, b=b):
                    pltpu.make_async_copy(
                        outbuf_v.at[k],
                        out_hbm.at[b, pl.ds(f0, _QF), pl.ds(nb0, _NBW), :],
                        sems[k],
                    ).wait()

                @plsc.parallel_loop(0, _NVEC)
                def _vec_body(j, k=k, b=b):
                    cvec = cidx_v[b, pl.ds(j * _L, _L)]
                    nb = j >> 3
                    no = (j & 7) * _L
                    for fi in range(_QF):
                        g = plsc.load_gather(ctab_v, [cvec + fi * _CT])
                        outbuf_v[k, fi, nb, pl.ds(no, _L)] = g

                pltpu.async_copy(
                    outbuf_v.at[k],
                    out_hbm.at[b, pl.ds(f0, _QF), pl.ds(nb0, _NBW), :],
                    sems[k],
                )
            return 0

        lax.fori_loop(0, _B // 2, batch_pair, 0)

        for k in range(2):
            pltpu.make_async_copy(
                outbuf_v.at[k],
                out_hbm.at[_B - 2 + k, pl.ds(f0, _QF), pl.ds(nb0, _NBW), :],
                sems[k],
            ).wait()


@functools.partial(
    pl.kernel,
    mesh=plsc.VectorSubcoreMesh(core_axis_name="c", subcore_axis_name="s"),
    out_type=jax.ShapeDtypeStruct((_B, _F, _NB, 128), jnp.float32),
    compiler_params=pltpu.CompilerParams(needs_layout_passes=False),
    scratch_types=[
        pltpu.VMEM((_B, _NPW), jnp.float32),
        pltpu.VMEM((_B, _NPW), jnp.int32),
        pltpu.VMEM((_F * _TIME,), jnp.float32),
        pltpu.VMEM((_F * _WPAD,), jnp.float32),
        pltpu.VMEM((_QF * _CT,), jnp.float32),
        pltpu.VMEM((2, _QF, _NBW, 128), jnp.float32),
        pltpu.SemaphoreType.DMA,
        pltpu.SemaphoreType.DMA,
    ],
)
def _sc_lookup(day_hbm, week_hbm, dayt_hbm, weekt_hbm, out_hbm,
               stage_v, cidx_v, dayt_v, weekt_v, ctab_v, outbuf_v,
               sem0, sem1):
    _tec_body(day_hbm, week_hbm, dayt_hbm, weekt_hbm, out_hbm,
              stage_v, cidx_v, dayt_v, weekt_v, ctab_v, outbuf_v,
              sem0, sem1)


def kernel(x, time_day, time_week):
    day_frac = x[:, _T - 1, :, 1]
    week_val = x[:, _T - 1, :, 2]
    dayt = jnp.transpose(time_day).reshape(-1)
    weekt = jnp.concatenate(
        [jnp.transpose(time_week),
         jnp.zeros((_F, _WPAD - 7), jnp.float32)], axis=1).reshape(-1)
    out = _sc_lookup(day_frac, week_val, dayt, weekt)
    return out.reshape(_B, _F, _N)[..., None]
